# baseline (speedup 1.0000x reference)
import functools

import jax
import jax.numpy as jnp
from jax.experimental import pallas as pl
from jax.experimental.pallas import tpu as pltpu

_B = 2
_N = 8192
_HI = 1e10
_PREC = jax.lax.Precision.HIGHEST


def _dot(a, b, prec=jax.lax.Precision.DEFAULT):
    return jax.lax.dot_general(a, b, (((1,), (0,)), ((), ())),
                               precision=prec, preferred_element_type=jnp.float32)


def _fps_body(x_ref, y_ref, z_ref, out_ref, *, npoint, n, b):
    x = x_ref[...]
    y = y_ref[...]
    z = z_ref[...]
    lane = jax.lax.broadcasted_iota(jnp.int32, (b, n), 1)

    def body(i, carry):
        dists, far = carry
        oh = (lane == far).astype(jnp.float32)
        cx = jnp.sum(x * oh, axis=1, keepdims=True)
        cy = jnp.sum(y * oh, axis=1, keepdims=True)
        cz = jnp.sum(z * oh, axis=1, keepdims=True)
        out_ref[pl.ds(i, 1), :, :] = jnp.concatenate([cx, cy, cz], axis=1)[None]
        d = (x - cx) ** 2 + (y - cy) ** 2 + (z - cz) ** 2
        dists = jnp.minimum(dists, d)
        m = jnp.max(dists, axis=1, keepdims=True)
        far = jnp.min(jnp.where(dists == m, lane, n), axis=1, keepdims=True)
        return dists, far

    init = (jnp.full((b, n), _HI, jnp.float32), jnp.zeros((b, 1), jnp.int32))
    jax.lax.fori_loop(0, npoint, body, init)


def _fps(xyzT, npoint):
    b, _, n = xyzT.shape
    x = xyzT[:, 0, :]
    y = xyzT[:, 1, :]
    z = xyzT[:, 2, :]
    out = pl.pallas_call(
        functools.partial(_fps_body, npoint=npoint, n=n, b=b),
        in_specs=[pl.BlockSpec((b, n), lambda: (0, 0))] * 3,
        out_specs=pl.BlockSpec((npoint, b, 3), lambda: (0, 0, 0)),
        out_shape=jax.ShapeDtypeStruct((npoint, b, 3), jnp.float32),
    )(x, y, z)
    return out.transpose(1, 0, 2)


def _select_body(xyzT_ref, tab_ref, ctr_ref, g_ref, d32_ref, dscr_ref, *,
                 n, cb, cp, k):
    x = xyzT_ref[0, 0:1, :]
    y = xyzT_ref[0, 1:2, :]
    z = xyzT_ref[0, 2:3, :]
    cx = ctr_ref[0, :, 0:1]
    cy = ctr_ref[0, :, 1:2]
    cz = ctr_ref[0, :, 2:3]
    dscr_ref[...] = (cx - x) ** 2 + (cy - y) ** 2 + (cz - z) ** 2
    lane = jax.lax.broadcasted_iota(jnp.int32, (cb, n), 1)
    col = jax.lax.broadcasted_iota(jnp.int32, (1, cp), 1)
    shift = (cx * (col == 0) + cy * (col == 1) + cz * (col == 2)).astype(jnp.float32)

    def step(s, _):
        d = dscr_ref[...]
        m = jnp.min(d, axis=1, keepdims=True)
        ohi = lane == idx_of(d, m)
        row = _dot(ohi.astype(jnp.float32), tab_ref[0], _PREC)
        g_ref[0, pl.ds(s, 1), :, :] = (row - shift)[None]
        d32_ref[0, pl.ds(s, 1), :, :] = m[None]
        dscr_ref[...] = jnp.where(ohi, _HI, d)
        return 0

    def idx_of(d, m):
        return jnp.min(jnp.where(d == m, lane, n), axis=1, keepdims=True)

    jax.lax.fori_loop(0, k, step, 0)


def _select(xyzT, tab, ctrs, k=32, cb=128):
    b, _, n = xyzT.shape
    m = ctrs.shape[1]
    cp = tab.shape[2]
    g, d32 = pl.pallas_call(
        functools.partial(_select_body, n=n, cb=cb, cp=cp, k=k),
        grid=(b, m // cb),
        in_specs=[
            pl.BlockSpec((1, 4, n), lambda i, j: (i, 0, 0)),
            pl.BlockSpec((1, n, cp), lambda i, j: (i, 0, 0)),
            pl.BlockSpec((1, cb, 3), lambda i, j: (i, j, 0)),
        ],
        out_specs=[
            pl.BlockSpec((1, k, cb, cp), lambda i, j: (i, 0, j, 0)),
            pl.BlockSpec((1, k, cb, 1), lambda i, j: (i, 0, j, 0)),
        ],
        out_shape=[
            jax.ShapeDtypeStruct((b, k, m, cp), jnp.float32),
            jax.ShapeDtypeStruct((b, k, m, 1), jnp.float32),
        ],
        scratch_shapes=[pltpu.VMEM((cb, n), jnp.float32)],
        compiler_params=pltpu.CompilerParams(
            dimension_semantics=("parallel", "parallel")),
    )(xyzT, tab, ctrs)
    return g, d32


def _mlpmax_body(*refs, nl, ns, rb, cp, r2):
    g_ref, d_ref = refs[0], refs[1]
    wb = refs[2:2 + 2 * nl]
    o_ref = refs[2 + 2 * nl]
    h = g_ref[0].reshape(ns * rb, cp)
    for i in range(nl):
        h = _dot(h, wb[2 * i][...]) + wb[2 * i + 1][...]
        h = jnp.maximum(h, 0.0)
    c = h.shape[-1]
    dflat = d_ref[0].reshape(ns * rb, 1)
    slot = jax.lax.broadcasted_iota(jnp.int32, (ns * rb, 1), 0)
    valid = (dflat <= r2) | (slot < rb)
    h = jnp.where(valid, h, -_HI)
    o_ref[0] = jnp.max(h.reshape(ns, rb, c), axis=0)


def _mlpmax(g, d32, layers, cin, r2, rb=128):
    b, ns, m, cp = g.shape
    dims = [cp] + [l["W"].shape[1] for l in layers]
    wbs = []
    for i, l in enumerate(layers):
        w = l["W"]
        if i == 0 and w.shape[0] != cp:
            w = jnp.zeros((cp, w.shape[1]), jnp.float32).at[:w.shape[0]].set(w)
        wbs.append(w)
        wbs.append(l["b"][None, :])
    nl = len(layers)
    in_specs = [
        pl.BlockSpec((1, ns, rb, cp), lambda i, j: (i, 0, j, 0)),
        pl.BlockSpec((1, ns, rb, 1), lambda i, j: (i, 0, j, 0)),
    ]
    for i in range(nl):
        in_specs.append(pl.BlockSpec((dims[i], dims[i + 1]), lambda i, j: (0, 0)))
        in_specs.append(pl.BlockSpec((1, dims[i + 1]), lambda i, j: (0, 0)))
    out = pl.pallas_call(
        functools.partial(_mlpmax_body, nl=nl, ns=ns, rb=rb, cp=cp, r2=r2),
        grid=(b, m // rb),
        in_specs=in_specs,
        out_specs=pl.BlockSpec((1, rb, dims[-1]), lambda i, j: (i, j, 0)),
        out_shape=jax.ShapeDtypeStruct((b, m, dims[-1]), jnp.float32),
        compiler_params=pltpu.CompilerParams(
            dimension_semantics=("parallel", "parallel")),
    )(g, d32, *wbs)
    return out


def _mlp_body(*refs, nl, last_linear):
    x_ref = refs[0]
    wb = refs[1:1 + 2 * nl]
    o_ref = refs[1 + 2 * nl]
    h = x_ref[...]
    for i in range(nl):
        h = _dot(h, wb[2 * i][...]) + wb[2 * i + 1][...]
        if not (last_linear and i == nl - 1):
            h = jnp.maximum(h, 0.0)
    o_ref[...] = h


def _mlp(x, layers, last_linear=False, rb=512):
    rows, cin = x.shape
    rb = min(rb, rows)
    dims = [cin] + [l["W"].shape[1] for l in layers]
    wbs = []
    for l in layers:
        wbs.append(l["W"])
        wbs.append(l["b"][None, :])
    nl = len(layers)
    in_specs = [pl.BlockSpec((rb, cin), lambda i: (i, 0))]
    for i in range(nl):
        in_specs.append(pl.BlockSpec((dims[i], dims[i + 1]), lambda i: (0, 0)))
        in_specs.append(pl.BlockSpec((1, dims[i + 1]), lambda i: (0, 0)))
    return pl.pallas_call(
        functools.partial(_mlp_body, nl=nl, last_linear=last_linear),
        grid=(rows // rb,),
        in_specs=in_specs,
        out_specs=pl.BlockSpec((rb, dims[-1]), lambda i: (i, 0)),
        out_shape=jax.ShapeDtypeStruct((rows, dims[-1]), jnp.float32),
        compiler_params=pltpu.CompilerParams(
            dimension_semantics=("parallel",)),
    )(x, *wbs)


def _topk_body(score_ref, x_ref, y_ref, z_ref, out_ref, *, m_in, k, b):
    x = x_ref[...]
    y = y_ref[...]
    z = z_ref[...]
    lane = jax.lax.broadcasted_iota(jnp.int32, (b, m_in), 1)

    def body(i, s):
        m = jnp.max(s, axis=1, keepdims=True)
        idx = jnp.min(jnp.where(s == m, lane, m_in), axis=1, keepdims=True)
        oh = (lane == idx).astype(jnp.float32)
        cx = jnp.sum(x * oh, axis=1, keepdims=True)
        cy = jnp.sum(y * oh, axis=1, keepdims=True)
        cz = jnp.sum(z * oh, axis=1, keepdims=True)
        out_ref[pl.ds(i, 1), :, :] = jnp.concatenate([cx, cy, cz], axis=1)[None]
        return jnp.where(lane == idx, -_HI, s)

    jax.lax.fori_loop(0, k, body, score_ref[...])


def _topk_gather(score, xyzT, k):
    b, m_in = score.shape
    out = pl.pallas_call(
        functools.partial(_topk_body, m_in=m_in, k=k, b=b),
        in_specs=[pl.BlockSpec((b, m_in), lambda: (0, 0))] * 4,
        out_specs=pl.BlockSpec((k, b, 3), lambda: (0, 0, 0)),
        out_shape=jax.ShapeDtypeStruct((k, b, 3), jnp.float32),
    )(score, xyzT[:, 0, :], xyzT[:, 1, :], xyzT[:, 2, :])
    return out.transpose(1, 0, 2)


def _vote_body(f_ref, xyz_ref, w_ref, b_ref, wo_ref, bo_ref, off_ref, ctr_ref):
    vf = jnp.maximum(_dot(f_ref[...], w_ref[...]) + b_ref[...], 0.0)
    off = _dot(vf, wo_ref[...]) + bo_ref[...]
    col3 = jax.lax.broadcasted_iota(jnp.int32, (1, 3), 1)
    mt = 3.0 - (col3 == 2).astype(jnp.float32)
    off = jnp.clip(off, -mt, mt)
    off_ref[...] = off
    ctr_ref[...] = xyz_ref[...] + off


def _vote(f3, xyz3, vote_mlp, vote_off):
    rows, c = f3.shape
    h = vote_mlp[0]["W"].shape[1]
    return pl.pallas_call(
        _vote_body,
        grid=(1,),
        in_specs=[
            pl.BlockSpec((rows, c), lambda i: (0, 0)),
            pl.BlockSpec((rows, 3), lambda i: (0, 0)),
            pl.BlockSpec((c, h), lambda i: (0, 0)),
            pl.BlockSpec((1, h), lambda i: (0, 0)),
            pl.BlockSpec((h, 3), lambda i: (0, 0)),
            pl.BlockSpec((1, 3), lambda i: (0, 0)),
        ],
        out_specs=[
            pl.BlockSpec((rows, 3), lambda i: (0, 0)),
            pl.BlockSpec((rows, 3), lambda i: (0, 0)),
        ],
        out_shape=[
            jax.ShapeDtypeStruct((rows, 3), jnp.float32),
            jax.ShapeDtypeStruct((rows, 3), jnp.float32),
        ],
    )(f3, xyz3, vote_mlp[0]["W"], vote_mlp[0]["b"][None, :],
      vote_off["W"], vote_off["b"][None, :])


def _to_T(xyz):
    b, n, _ = xyz.shape
    pad = jnp.zeros((b, n, 1), jnp.float32)
    return jnp.concatenate([xyz, pad], axis=-1).transpose(0, 2, 1)


def _table(xyz, feats, cp):
    b, n, _ = xyz.shape
    c = feats.shape[-1]
    pad = jnp.zeros((b, n, cp - 3 - c), jnp.float32)
    return jnp.concatenate([xyz, feats, pad], axis=-1)


def _sa(params, xyzT, tab, ctrs, r2s, nsamples, cins):
    g, d32 = _select(xyzT, tab, ctrs, k=32)
    outs = []
    for s in range(2):
        ns = nsamples[s]
        h = _mlpmax(g[:, :ns], d32[:, :ns], params["scale%d" % s],
                    cin=cins[s], r2=r2s[s])
        outs.append(h)
    b, m, _ = outs[0].shape
    cat = jnp.concatenate(outs, axis=-1).reshape(b * m, -1)
    f = _mlp(cat, params["agg"])
    return f.reshape(b, m, f.shape[-1])


def kernel(points, params):
    pts = points.reshape(_B, _N, 5)
    xyz = pts[:, :, 1:4]
    feats = pts[:, :, 4:5]

    xyzT = _to_T(xyz)
    tab1 = _table(xyz, feats, 8)
    xyz1 = _fps(xyzT, 1024)
    f1 = _sa(params["sa1"], xyzT, tab1, xyz1, [0.25, 1.0], [16, 32], [4, 4])

    xyz1T = _to_T(xyz1)
    tab2 = _table(xyz1, f1, 72)
    xyz2 = _fps(xyz1T, 512)
    f2 = _sa(params["sa2"], xyz1T, tab2, xyz2, [1.0, 4.0], [16, 32], [67, 67])

    cls2 = _mlp(f2.reshape(-1, f2.shape[-1]), params["conf2"], last_linear=True)
    cls2 = cls2.reshape(_B, 512, 3)

    xyz2T = _to_T(xyz2)
    score = jnp.max(jax.nn.sigmoid(cls2), axis=-1)
    xyz3 = _topk_gather(score, xyz2T, 256)

    tab3 = _table(xyz2, f2, 136)
    f3 = _sa(params["sa3"], xyz2T, tab3, xyz3, [4.0, 16.0], [16, 32], [131, 131])

    cls3 = _mlp(f3.reshape(-1, f3.shape[-1]), params["conf3"], last_linear=True)
    cls3 = cls3.reshape(_B, 256, 3)

    offsets, centers = _vote(f3.reshape(-1, f3.shape[-1]),
                             xyz3.reshape(-1, 3),
                             params["vote_mlp"], params["vote_off"])
    offsets = offsets.reshape(_B, 256, 3)
    centers = centers.reshape(_B, 256, 3)

    xyz3T = _to_T(xyz3)
    tab4 = _table(xyz3, f3, 264)
    f4 = _sa(params["sa4"], xyz3T, tab4, centers, [16.0, 64.0], [16, 32], [259, 259])

    ctr_b = pts[:, :256, 0].reshape(-1, 1)
    centers_out = jnp.concatenate([ctr_b, centers.reshape(-1, 3)], axis=1)
    centers_origin_out = jnp.concatenate([ctr_b, xyz3.reshape(-1, 3)], axis=1)
    ctr_offsets_out = jnp.concatenate([ctr_b, offsets.reshape(-1, 3)], axis=1)
    centers_features = f4.reshape(-1, f4.shape[-1])
    return (centers_out, centers_origin_out, ctr_offsets_out,
            centers_features, cls2, cls3)


# speedup vs baseline: 1.8387x; 1.8387x over previous
import functools

import jax
import jax.numpy as jnp
from jax.experimental import pallas as pl
from jax.experimental.pallas import tpu as pltpu
from jax.experimental.pallas import tpu_sc as plsc

_B = 2
_N = 8192
_HI = 1e10
_PREC = jax.lax.Precision.HIGHEST


def _dot(a, b, prec=jax.lax.Precision.DEFAULT):
    return jax.lax.dot_general(a, b, (((1,), (0,)), ((), ())),
                               precision=prec, preferred_element_type=jnp.float32)


def _fps_body(x_ref, y_ref, z_ref, out_ref, *, npoint, n, b):
    x = x_ref[...]
    y = y_ref[...]
    z = z_ref[...]
    lane = jax.lax.broadcasted_iota(jnp.int32, (b, n), 1)

    def body(i, carry):
        dists, far = carry
        oh = (lane == far).astype(jnp.float32)
        cx = jnp.sum(x * oh, axis=1, keepdims=True)
        cy = jnp.sum(y * oh, axis=1, keepdims=True)
        cz = jnp.sum(z * oh, axis=1, keepdims=True)
        out_ref[pl.ds(i, 1), :, :] = jnp.concatenate([cx, cy, cz], axis=1)[None]
        d = (x - cx) ** 2 + (y - cy) ** 2 + (z - cz) ** 2
        dists = jnp.minimum(dists, d)
        m = jnp.max(dists, axis=1, keepdims=True)
        far = jnp.min(jnp.where(dists == m, lane, n), axis=1, keepdims=True)
        return dists, far

    init = (jnp.full((b, n), _HI, jnp.float32), jnp.zeros((b, 1), jnp.int32))
    jax.lax.fori_loop(0, npoint, body, init)


def _fps(xyzT, npoint):
    b, _, n = xyzT.shape
    x = xyzT[:, 0, :]
    y = xyzT[:, 1, :]
    z = xyzT[:, 2, :]
    out = pl.pallas_call(
        functools.partial(_fps_body, npoint=npoint, n=n, b=b),
        in_specs=[pl.BlockSpec((b, n), lambda: (0, 0))] * 3,
        out_specs=pl.BlockSpec((npoint, b, 3), lambda: (0, 0, 0)),
        out_shape=jax.ShapeDtypeStruct((npoint, b, 3), jnp.float32),
    )(x, y, z)
    return out.transpose(1, 0, 2)


def _select_body(xyzT_ref, ctr_ref, idx_ref, d32_ref, dscr_ref, *, n, cb, k):
    x = xyzT_ref[0, 0:1, :]
    y = xyzT_ref[0, 1:2, :]
    z = xyzT_ref[0, 2:3, :]
    cx = ctr_ref[0, :, 0:1]
    cy = ctr_ref[0, :, 1:2]
    cz = ctr_ref[0, :, 2:3]
    dscr_ref[...] = (cx - x) ** 2 + (cy - y) ** 2 + (cz - z) ** 2
    lane = jax.lax.broadcasted_iota(jnp.int32, (cb, n), 1)
    base = pl.program_id(0) * n

    def step(s, _):
        d = dscr_ref[...]
        m = jnp.min(d, axis=1, keepdims=True)
        idx = jnp.min(jnp.where(d == m, lane, n), axis=1, keepdims=True)
        idx_ref[0, pl.ds(s, 1), :, :] = (idx + base)[None]
        d32_ref[0, pl.ds(s, 1), :, :] = m[None]
        dscr_ref[...] = jnp.where(lane == idx, _HI, d)
        return 0

    jax.lax.fori_loop(0, k, step, 0)


def _select(xyzT, ctrs, k=32, cb=128):
    b, _, n = xyzT.shape
    m = ctrs.shape[1]
    idx, d32 = pl.pallas_call(
        functools.partial(_select_body, n=n, cb=cb, k=k),
        grid=(b, m // cb),
        in_specs=[
            pl.BlockSpec((1, 4, n), lambda i, j: (i, 0, 0)),
            pl.BlockSpec((1, cb, 3), lambda i, j: (i, j, 0)),
        ],
        out_specs=[
            pl.BlockSpec((1, k, cb, 1), lambda i, j: (i, 0, j, 0)),
            pl.BlockSpec((1, k, cb, 1), lambda i, j: (i, 0, j, 0)),
        ],
        out_shape=[
            jax.ShapeDtypeStruct((b, k, m, 1), jnp.int32),
            jax.ShapeDtypeStruct((b, k, m, 1), jnp.float32),
        ],
        scratch_shapes=[pltpu.VMEM((cb, n), jnp.float32)],
        compiler_params=pltpu.CompilerParams(
            dimension_semantics=("parallel", "parallel")),
    )(xyzT, ctrs)
    return idx, d32


def _sc_gather(tabf, idxf):
    v, dd = tabf.shape
    (bb,) = idxf.shape
    nc = 2
    nw = nc * 16
    bpw = bb // nw
    chunk = bpw
    while chunk * (dd + 1) > 120000:
        chunk //= 2
    nch = bpw // chunk
    mesh = plsc.VectorSubcoreMesh(core_axis_name="c", subcore_axis_name="s")

    @functools.partial(
        pl.kernel, mesh=mesh,
        out_type=jax.ShapeDtypeStruct((bb, dd), jnp.float32),
        scratch_types=[
            pltpu.VMEM((chunk,), jnp.int32),
            pltpu.VMEM((chunk, dd), jnp.float32),
            pltpu.SemaphoreType.DMA,
        ],
        compiler_params=pltpu.CompilerParams(use_tc_tiling_on_sc=False),
    )
    def k(table_hbm, idx_hbm, out_hbm, idx_v, rows_v, sem):
        wid = jax.lax.axis_index("s") * nc + jax.lax.axis_index("c")
        for ci in range(nch):
            base = wid * bpw + ci * chunk
            pltpu.sync_copy(idx_hbm.at[pl.ds(base, chunk)], idx_v)
            pltpu.async_copy(table_hbm.at[idx_v], rows_v, sem).wait()
            pltpu.sync_copy(rows_v, out_hbm.at[pl.ds(base, chunk)])

    return k(tabf, idxf)


def _mlpmax_body(*refs, nl, ns, rb, cp, r2):
    g_ref, d_ref, ctr_ref = refs[0], refs[1], refs[2]
    wb = refs[3:3 + 2 * nl]
    o_ref = refs[3 + 2 * nl]
    cx = ctr_ref[0][:, 0:1]
    cy = ctr_ref[0][:, 1:2]
    cz = ctr_ref[0][:, 2:3]
    col = jax.lax.broadcasted_iota(jnp.int32, (1, cp), 1)
    shift = (cx * (col == 0) + cy * (col == 1) + cz * (col == 2)).astype(jnp.float32)
    h = (g_ref[0] - shift[None]).reshape(ns * rb, cp)
    for i in range(nl):
        h = _dot(h, wb[2 * i][...]) + wb[2 * i + 1][...]
        h = jnp.maximum(h, 0.0)
    c = h.shape[-1]
    dflat = d_ref[0].reshape(ns * rb, 1)
    slot = jax.lax.broadcasted_iota(jnp.int32, (ns * rb, 1), 0)
    valid = (dflat <= r2) | (slot < rb)
    h = jnp.where(valid, h, -_HI)
    o_ref[0] = jnp.max(h.reshape(ns, rb, c), axis=0)


def _mlpmax(g, d32, ctrs, layers, r2, rb=128):
    b, ns, m, cp = g.shape
    dims = [cp] + [l["W"].shape[1] for l in layers]
    wbs = []
    for i, l in enumerate(layers):
        w = l["W"]
        if i == 0 and w.shape[0] != cp:
            w = jnp.zeros((cp, w.shape[1]), jnp.float32).at[:w.shape[0]].set(w)
        wbs.append(w)
        wbs.append(l["b"][None, :])
    nl = len(layers)
    in_specs = [
        pl.BlockSpec((1, ns, rb, cp), lambda i, j: (i, 0, j, 0)),
        pl.BlockSpec((1, ns, rb, 1), lambda i, j: (i, 0, j, 0)),
        pl.BlockSpec((1, rb, 3), lambda i, j: (i, j, 0)),
    ]
    for i in range(nl):
        in_specs.append(pl.BlockSpec((dims[i], dims[i + 1]), lambda i, j: (0, 0)))
        in_specs.append(pl.BlockSpec((1, dims[i + 1]), lambda i, j: (0, 0)))
    out = pl.pallas_call(
        functools.partial(_mlpmax_body, nl=nl, ns=ns, rb=rb, cp=cp, r2=r2),
        grid=(b, m // rb),
        in_specs=in_specs,
        out_specs=pl.BlockSpec((1, rb, dims[-1]), lambda i, j: (i, j, 0)),
        out_shape=jax.ShapeDtypeStruct((b, m, dims[-1]), jnp.float32),
        compiler_params=pltpu.CompilerParams(
            dimension_semantics=("parallel", "parallel")),
    )(g, d32, ctrs, *wbs)
    return out


def _mlp_body(*refs, nl, last_linear):
    x_ref = refs[0]
    wb = refs[1:1 + 2 * nl]
    o_ref = refs[1 + 2 * nl]
    h = x_ref[...]
    for i in range(nl):
        h = _dot(h, wb[2 * i][...]) + wb[2 * i + 1][...]
        if not (last_linear and i == nl - 1):
            h = jnp.maximum(h, 0.0)
    o_ref[...] = h


def _mlp(x, layers, last_linear=False, rb=512):
    rows, cin = x.shape
    rb = min(rb, rows)
    dims = [cin] + [l["W"].shape[1] for l in layers]
    wbs = []
    for l in layers:
        wbs.append(l["W"])
        wbs.append(l["b"][None, :])
    nl = len(layers)
    in_specs = [pl.BlockSpec((rb, cin), lambda i: (i, 0))]
    for i in range(nl):
        in_specs.append(pl.BlockSpec((dims[i], dims[i + 1]), lambda i: (0, 0)))
        in_specs.append(pl.BlockSpec((1, dims[i + 1]), lambda i: (0, 0)))
    return pl.pallas_call(
        functools.partial(_mlp_body, nl=nl, last_linear=last_linear),
        grid=(rows // rb,),
        in_specs=in_specs,
        out_specs=pl.BlockSpec((rb, dims[-1]), lambda i: (i, 0)),
        out_shape=jax.ShapeDtypeStruct((rows, dims[-1]), jnp.float32),
        compiler_params=pltpu.CompilerParams(
            dimension_semantics=("parallel",)),
    )(x, *wbs)


def _topk_body(score_ref, x_ref, y_ref, z_ref, out_ref, *, m_in, k, b):
    x = x_ref[...]
    y = y_ref[...]
    z = z_ref[...]
    lane = jax.lax.broadcasted_iota(jnp.int32, (b, m_in), 1)

    def body(i, s):
        m = jnp.max(s, axis=1, keepdims=True)
        idx = jnp.min(jnp.where(s == m, lane, m_in), axis=1, keepdims=True)
        oh = (lane == idx).astype(jnp.float32)
        cx = jnp.sum(x * oh, axis=1, keepdims=True)
        cy = jnp.sum(y * oh, axis=1, keepdims=True)
        cz = jnp.sum(z * oh, axis=1, keepdims=True)
        out_ref[pl.ds(i, 1), :, :] = jnp.concatenate([cx, cy, cz], axis=1)[None]
        return jnp.where(lane == idx, -_HI, s)

    jax.lax.fori_loop(0, k, body, score_ref[...])


def _topk_gather(score, xyzT, k):
    b, m_in = score.shape
    out = pl.pallas_call(
        functools.partial(_topk_body, m_in=m_in, k=k, b=b),
        in_specs=[pl.BlockSpec((b, m_in), lambda: (0, 0))] * 4,
        out_specs=pl.BlockSpec((k, b, 3), lambda: (0, 0, 0)),
        out_shape=jax.ShapeDtypeStruct((k, b, 3), jnp.float32),
    )(score, xyzT[:, 0, :], xyzT[:, 1, :], xyzT[:, 2, :])
    return out.transpose(1, 0, 2)


def _vote_body(f_ref, xyz_ref, w_ref, b_ref, wo_ref, bo_ref, off_ref, ctr_ref):
    vf = jnp.maximum(_dot(f_ref[...], w_ref[...]) + b_ref[...], 0.0)
    off = _dot(vf, wo_ref[...]) + bo_ref[...]
    col3 = jax.lax.broadcasted_iota(jnp.int32, (1, 3), 1)
    mt = 3.0 - (col3 == 2).astype(jnp.float32)
    off = jnp.clip(off, -mt, mt)
    off_ref[...] = off
    ctr_ref[...] = xyz_ref[...] + off


def _vote(f3, xyz3, vote_mlp, vote_off):
    rows, c = f3.shape
    h = vote_mlp[0]["W"].shape[1]
    return pl.pallas_call(
        _vote_body,
        grid=(1,),
        in_specs=[
            pl.BlockSpec((rows, c), lambda i: (0, 0)),
            pl.BlockSpec((rows, 3), lambda i: (0, 0)),
            pl.BlockSpec((c, h), lambda i: (0, 0)),
            pl.BlockSpec((1, h), lambda i: (0, 0)),
            pl.BlockSpec((h, 3), lambda i: (0, 0)),
            pl.BlockSpec((1, 3), lambda i: (0, 0)),
        ],
        out_specs=[
            pl.BlockSpec((rows, 3), lambda i: (0, 0)),
            pl.BlockSpec((rows, 3), lambda i: (0, 0)),
        ],
        out_shape=[
            jax.ShapeDtypeStruct((rows, 3), jnp.float32),
            jax.ShapeDtypeStruct((rows, 3), jnp.float32),
        ],
    )(f3, xyz3, vote_mlp[0]["W"], vote_mlp[0]["b"][None, :],
      vote_off["W"], vote_off["b"][None, :])


def _to_T(xyz):
    b, n, _ = xyz.shape
    pad = jnp.zeros((b, n, 1), jnp.float32)
    return jnp.concatenate([xyz, pad], axis=-1).transpose(0, 2, 1)


def _table(xyz, feats, cp):
    b, n, _ = xyz.shape
    c = feats.shape[-1]
    pad = jnp.zeros((b, n, cp - 3 - c), jnp.float32)
    return jnp.concatenate([xyz, feats, pad], axis=-1).reshape(b * n, cp)


def _sa(params, xyzT, tabf, ctrs, r2s, nsamples):
    idx, d32 = _select(xyzT, ctrs, k=32)
    b, k, m, _ = idx.shape
    rows = _sc_gather(tabf, idx.reshape(-1))
    g = rows.reshape(b, k, m, rows.shape[-1])
    outs = []
    for s in range(2):
        ns = nsamples[s]
        h = _mlpmax(g[:, :ns], d32[:, :ns], ctrs, params["scale%d" % s],
                    r2=r2s[s])
        outs.append(h)
    b, m, _ = outs[0].shape
    cat = jnp.concatenate(outs, axis=-1).reshape(b * m, -1)
    f = _mlp(cat, params["agg"])
    return f.reshape(b, m, f.shape[-1])


def kernel(points, params):
    pts = points.reshape(_B, _N, 5)
    xyz = pts[:, :, 1:4]
    feats = pts[:, :, 4:5]

    xyzT = _to_T(xyz)
    tab1 = _table(xyz, feats, 16)
    xyz1 = _fps(xyzT, 1024)
    f1 = _sa(params["sa1"], xyzT, tab1, xyz1, [0.25, 1.0], [16, 32])

    xyz1T = _to_T(xyz1)
    tab2 = _table(xyz1, f1, 80)
    xyz2 = _fps(xyz1T, 512)
    f2 = _sa(params["sa2"], xyz1T, tab2, xyz2, [1.0, 4.0], [16, 32])

    cls2 = _mlp(f2.reshape(-1, f2.shape[-1]), params["conf2"], last_linear=True)
    cls2 = cls2.reshape(_B, 512, 3)

    xyz2T = _to_T(xyz2)
    score = jnp.max(jax.nn.sigmoid(cls2), axis=-1)
    xyz3 = _topk_gather(score, xyz2T, 256)

    tab3 = _table(xyz2, f2, 144)
    f3 = _sa(params["sa3"], xyz2T, tab3, xyz3, [4.0, 16.0], [16, 32])

    cls3 = _mlp(f3.reshape(-1, f3.shape[-1]), params["conf3"], last_linear=True)
    cls3 = cls3.reshape(_B, 256, 3)

    offsets, centers = _vote(f3.reshape(-1, f3.shape[-1]),
                             xyz3.reshape(-1, 3),
                             params["vote_mlp"], params["vote_off"])
    offsets = offsets.reshape(_B, 256, 3)
    centers = centers.reshape(_B, 256, 3)

    xyz3T = _to_T(xyz3)
    tab4 = _table(xyz3, f3, 272)
    f4 = _sa(params["sa4"], xyz3T, tab4, centers, [16.0, 64.0], [16, 32])

    ctr_b = pts[:, :256, 0].reshape(-1, 1)
    centers_out = jnp.concatenate([ctr_b, centers.reshape(-1, 3)], axis=1)
    centers_origin_out = jnp.concatenate([ctr_b, xyz3.reshape(-1, 3)], axis=1)
    ctr_offsets_out = jnp.concatenate([ctr_b, offsets.reshape(-1, 3)], axis=1)
    centers_features = f4.reshape(-1, f4.shape[-1])
    return (centers_out, centers_origin_out, ctr_offsets_out,
            centers_features, cls2, cls3)


# speedup vs baseline: 1.9416x; 1.0560x over previous
import functools

import jax
import jax.numpy as jnp
from jax.experimental import pallas as pl
from jax.experimental.pallas import tpu as pltpu
from jax.experimental.pallas import tpu_sc as plsc

_B = 2
_N = 8192
_HI = 1e10
_PREC = jax.lax.Precision.HIGHEST


def _dot(a, b, prec=jax.lax.Precision.DEFAULT):
    return jax.lax.dot_general(a, b, (((1,), (0,)), ((), ())),
                               precision=prec, preferred_element_type=jnp.float32)


def _fps_body(x_ref, y_ref, z_ref, out_ref, *, npoint, n, b):
    x = x_ref[...]
    y = y_ref[...]
    z = z_ref[...]
    lane = jax.lax.broadcasted_iota(jnp.int32, (b, n), 1)

    def body(i, carry):
        dists, far = carry
        oh = (lane == far).astype(jnp.float32)
        cx = jnp.sum(x * oh, axis=1, keepdims=True)
        cy = jnp.sum(y * oh, axis=1, keepdims=True)
        cz = jnp.sum(z * oh, axis=1, keepdims=True)
        out_ref[pl.ds(i, 1), :, :] = jnp.concatenate([cx, cy, cz], axis=1)[None]
        d = (x - cx) ** 2 + (y - cy) ** 2 + (z - cz) ** 2
        dists = jnp.minimum(dists, d)
        m = jnp.max(dists, axis=1, keepdims=True)
        far = jnp.min(jnp.where(dists == m, lane, n), axis=1, keepdims=True)
        return dists, far

    init = (jnp.full((b, n), _HI, jnp.float32), jnp.zeros((b, 1), jnp.int32))
    jax.lax.fori_loop(0, npoint, body, init)


def _fps(xyzT, npoint):
    b, _, n = xyzT.shape
    x = xyzT[:, 0, :]
    y = xyzT[:, 1, :]
    z = xyzT[:, 2, :]
    out = pl.pallas_call(
        functools.partial(_fps_body, npoint=npoint, n=n, b=b),
        in_specs=[pl.BlockSpec((b, n), lambda: (0, 0))] * 3,
        out_specs=pl.BlockSpec((npoint, b, 3), lambda: (0, 0, 0)),
        out_shape=jax.ShapeDtypeStruct((npoint, b, 3), jnp.float32),
    )(x, y, z)
    return out.transpose(1, 0, 2)


def _select_body(xyzT_ref, ctr_ref, idx_ref, d32_ref, dscr_ref, *, n, cb, k):
    x = xyzT_ref[0, 0:1, :]
    y = xyzT_ref[0, 1:2, :]
    z = xyzT_ref[0, 2:3, :]
    cx = ctr_ref[0, :, 0:1]
    cy = ctr_ref[0, :, 1:2]
    cz = ctr_ref[0, :, 2:3]
    dscr_ref[...] = (cx - x) ** 2 + (cy - y) ** 2 + (cz - z) ** 2
    lane = jax.lax.broadcasted_iota(jnp.int32, (cb, n), 1)
    base = pl.program_id(0) * n

    def step(s, _):
        d = dscr_ref[...]
        m = jnp.min(d, axis=1, keepdims=True)
        idx = jnp.min(jnp.where(d == m, lane, n), axis=1, keepdims=True)
        idx_ref[0, pl.ds(s, 1), :, :] = (idx + base)[None]
        d32_ref[0, pl.ds(s, 1), :, :] = m[None]
        dscr_ref[...] = jnp.where(lane == idx, _HI, d)
        return 0

    jax.lax.fori_loop(0, k, step, 0)


def _select(xyzT, ctrs, k=32, cb=128):
    b, _, n = xyzT.shape
    m = ctrs.shape[1]
    idx, d32 = pl.pallas_call(
        functools.partial(_select_body, n=n, cb=cb, k=k),
        grid=(b, m // cb),
        in_specs=[
            pl.BlockSpec((1, 4, n), lambda i, j: (i, 0, 0)),
            pl.BlockSpec((1, cb, 3), lambda i, j: (i, j, 0)),
        ],
        out_specs=[
            pl.BlockSpec((1, k, cb, 1), lambda i, j: (i, 0, j, 0)),
            pl.BlockSpec((1, k, cb, 1), lambda i, j: (i, 0, j, 0)),
        ],
        out_shape=[
            jax.ShapeDtypeStruct((b, k, m, 1), jnp.int32),
            jax.ShapeDtypeStruct((b, k, m, 1), jnp.float32),
        ],
        scratch_shapes=[pltpu.VMEM((cb, n), jnp.float32)],
        compiler_params=pltpu.CompilerParams(
            dimension_semantics=("parallel", "parallel")),
    )(xyzT, ctrs)
    return idx, d32


def _sc_gather(tabf, idxf):
    v, dd = tabf.shape
    (bb,) = idxf.shape
    nc = 2
    nw = nc * 16
    bpw = bb // nw
    chunk = bpw
    while chunk * (dd + 1) > 120000:
        chunk //= 2
    nch = bpw // chunk
    mesh = plsc.VectorSubcoreMesh(core_axis_name="c", subcore_axis_name="s")

    @functools.partial(
        pl.kernel, mesh=mesh,
        out_type=jax.ShapeDtypeStruct((bb, dd), jnp.float32),
        scratch_types=[
            pltpu.VMEM((chunk,), jnp.int32),
            pltpu.VMEM((chunk, dd), jnp.float32),
            pltpu.SemaphoreType.DMA,
        ],
        compiler_params=pltpu.CompilerParams(use_tc_tiling_on_sc=False),
    )
    def k(table_hbm, idx_hbm, out_hbm, idx_v, rows_v, sem):
        wid = jax.lax.axis_index("s") * nc + jax.lax.axis_index("c")
        for ci in range(nch):
            base = wid * bpw + ci * chunk
            pltpu.sync_copy(idx_hbm.at[pl.ds(base, chunk)], idx_v)
            pltpu.async_copy(table_hbm.at[idx_v], rows_v, sem).wait()
            pltpu.sync_copy(rows_v, out_hbm.at[pl.ds(base, chunk)])

    return k(tabf, idxf)


def _maxed_mlp(x, d, wb, rb, r2):
    ns_rb = x.shape[0]
    h = x
    for i in range(len(wb) // 2):
        h = _dot(h, wb[2 * i][...]) + wb[2 * i + 1][...]
        h = jnp.maximum(h, 0.0)
    c = h.shape[-1]
    slot = jax.lax.broadcasted_iota(jnp.int32, (ns_rb, 1), 0)
    valid = (d <= r2) | (slot < rb)
    h = jnp.where(valid, h, -_HI)
    return jnp.max(h.reshape(ns_rb // rb, rb, c), axis=0)


def _mlpmax_body(*refs, nl0, nl1, ns, rb, cp, r20, r21):
    g_ref, d_ref, ctr_ref = refs[0], refs[1], refs[2]
    wb0 = refs[3:3 + 2 * nl0]
    wb1 = refs[3 + 2 * nl0:3 + 2 * nl0 + 2 * nl1]
    wa, ba = refs[3 + 2 * nl0 + 2 * nl1:3 + 2 * nl0 + 2 * nl1 + 2]
    o_ref = refs[-1]
    cx = ctr_ref[0][:, 0:1]
    cy = ctr_ref[0][:, 1:2]
    cz = ctr_ref[0][:, 2:3]
    col = jax.lax.broadcasted_iota(jnp.int32, (1, cp), 1)
    shift = (cx * (col == 0) + cy * (col == 1) + cz * (col == 2)).astype(jnp.float32)
    x = (g_ref[0] - shift[None]).reshape(ns * rb, cp)
    d = d_ref[0].reshape(ns * rb, 1)
    o0 = _maxed_mlp(x[:16 * rb], d[:16 * rb], wb0, rb, r20)
    o1 = _maxed_mlp(x, d, wb1, rb, r21)
    cat = jnp.concatenate([o0, o1], axis=1)
    o_ref[0] = jnp.maximum(_dot(cat, wa[...]) + ba[...], 0.0)


def _pad_first(layers, cp):
    wbs = []
    for i, l in enumerate(layers):
        w = l["W"]
        if i == 0 and w.shape[0] != cp:
            w = jnp.zeros((cp, w.shape[1]), jnp.float32).at[:w.shape[0]].set(w)
        wbs.append(w)
        wbs.append(l["b"][None, :])
    return wbs


def _mlpmax(g, d32, ctrs, params, r20, r21, rb=128):
    b, ns, m, cp = g.shape
    wb0 = _pad_first(params["scale0"], cp)
    wb1 = _pad_first(params["scale1"], cp)
    wba = [params["agg"][0]["W"], params["agg"][0]["b"][None, :]]
    nl0 = len(params["scale0"])
    nl1 = len(params["scale1"])
    in_specs = [
        pl.BlockSpec((1, ns, rb, cp), lambda i, j: (i, 0, j, 0)),
        pl.BlockSpec((1, ns, rb, 1), lambda i, j: (i, 0, j, 0)),
        pl.BlockSpec((1, rb, 3), lambda i, j: (i, j, 0)),
    ]
    for w in wb0 + wb1 + wba:
        in_specs.append(
            pl.BlockSpec(w.shape, lambda i, j, r=len(w.shape): (0,) * r))
    cagg = wba[0].shape[1]
    out = pl.pallas_call(
        functools.partial(_mlpmax_body, nl0=nl0, nl1=nl1, ns=ns, rb=rb, cp=cp,
                          r20=r20, r21=r21),
        grid=(b, m // rb),
        in_specs=in_specs,
        out_specs=pl.BlockSpec((1, rb, cagg), lambda i, j: (i, j, 0)),
        out_shape=jax.ShapeDtypeStruct((b, m, cagg), jnp.float32),
        compiler_params=pltpu.CompilerParams(
            dimension_semantics=("parallel", "parallel")),
    )(g, d32, ctrs, *wb0, *wb1, *wba)
    return out


def _mlp_body(*refs, nl, last_linear):
    x_ref = refs[0]
    wb = refs[1:1 + 2 * nl]
    o_ref = refs[1 + 2 * nl]
    h = x_ref[...]
    for i in range(nl):
        h = _dot(h, wb[2 * i][...]) + wb[2 * i + 1][...]
        if not (last_linear and i == nl - 1):
            h = jnp.maximum(h, 0.0)
    o_ref[...] = h


def _mlp(x, layers, last_linear=False, rb=512):
    rows, cin = x.shape
    rb = min(rb, rows)
    dims = [cin] + [l["W"].shape[1] for l in layers]
    wbs = []
    for l in layers:
        wbs.append(l["W"])
        wbs.append(l["b"][None, :])
    nl = len(layers)
    in_specs = [pl.BlockSpec((rb, cin), lambda i: (i, 0))]
    for i in range(nl):
        in_specs.append(pl.BlockSpec((dims[i], dims[i + 1]), lambda i: (0, 0)))
        in_specs.append(pl.BlockSpec((1, dims[i + 1]), lambda i: (0, 0)))
    return pl.pallas_call(
        functools.partial(_mlp_body, nl=nl, last_linear=last_linear),
        grid=(rows // rb,),
        in_specs=in_specs,
        out_specs=pl.BlockSpec((rb, dims[-1]), lambda i: (i, 0)),
        out_shape=jax.ShapeDtypeStruct((rows, dims[-1]), jnp.float32),
        compiler_params=pltpu.CompilerParams(
            dimension_semantics=("parallel",)),
    )(x, *wbs)


def _topk_body(score_ref, x_ref, y_ref, z_ref, out_ref, *, m_in, k, b):
    x = x_ref[...]
    y = y_ref[...]
    z = z_ref[...]
    lane = jax.lax.broadcasted_iota(jnp.int32, (b, m_in), 1)

    def body(i, s):
        m = jnp.max(s, axis=1, keepdims=True)
        idx = jnp.min(jnp.where(s == m, lane, m_in), axis=1, keepdims=True)
        oh = (lane == idx).astype(jnp.float32)
        cx = jnp.sum(x * oh, axis=1, keepdims=True)
        cy = jnp.sum(y * oh, axis=1, keepdims=True)
        cz = jnp.sum(z * oh, axis=1, keepdims=True)
        out_ref[pl.ds(i, 1), :, :] = jnp.concatenate([cx, cy, cz], axis=1)[None]
        return jnp.where(lane == idx, -_HI, s)

    jax.lax.fori_loop(0, k, body, score_ref[...])


def _topk_gather(score, xyzT, k):
    b, m_in = score.shape
    out = pl.pallas_call(
        functools.partial(_topk_body, m_in=m_in, k=k, b=b),
        in_specs=[pl.BlockSpec((b, m_in), lambda: (0, 0))] * 4,
        out_specs=pl.BlockSpec((k, b, 3), lambda: (0, 0, 0)),
        out_shape=jax.ShapeDtypeStruct((k, b, 3), jnp.float32),
    )(score, xyzT[:, 0, :], xyzT[:, 1, :], xyzT[:, 2, :])
    return out.transpose(1, 0, 2)


def _vote_body(f_ref, xyz_ref, w_ref, b_ref, wo_ref, bo_ref, off_ref, ctr_ref):
    vf = jnp.maximum(_dot(f_ref[...], w_ref[...]) + b_ref[...], 0.0)
    off = _dot(vf, wo_ref[...]) + bo_ref[...]
    col3 = jax.lax.broadcasted_iota(jnp.int32, (1, 3), 1)
    mt = 3.0 - (col3 == 2).astype(jnp.float32)
    off = jnp.clip(off, -mt, mt)
    off_ref[...] = off
    ctr_ref[...] = xyz_ref[...] + off


def _vote(f3, xyz3, vote_mlp, vote_off):
    rows, c = f3.shape
    h = vote_mlp[0]["W"].shape[1]
    return pl.pallas_call(
        _vote_body,
        grid=(1,),
        in_specs=[
            pl.BlockSpec((rows, c), lambda i: (0, 0)),
            pl.BlockSpec((rows, 3), lambda i: (0, 0)),
            pl.BlockSpec((c, h), lambda i: (0, 0)),
            pl.BlockSpec((1, h), lambda i: (0, 0)),
            pl.BlockSpec((h, 3), lambda i: (0, 0)),
            pl.BlockSpec((1, 3), lambda i: (0, 0)),
        ],
        out_specs=[
            pl.BlockSpec((rows, 3), lambda i: (0, 0)),
            pl.BlockSpec((rows, 3), lambda i: (0, 0)),
        ],
        out_shape=[
            jax.ShapeDtypeStruct((rows, 3), jnp.float32),
            jax.ShapeDtypeStruct((rows, 3), jnp.float32),
        ],
    )(f3, xyz3, vote_mlp[0]["W"], vote_mlp[0]["b"][None, :],
      vote_off["W"], vote_off["b"][None, :])


def _to_T(xyz):
    b, n, _ = xyz.shape
    pad = jnp.zeros((b, n, 1), jnp.float32)
    return jnp.concatenate([xyz, pad], axis=-1).transpose(0, 2, 1)


def _table(xyz, feats, cp):
    b, n, _ = xyz.shape
    c = feats.shape[-1]
    pad = jnp.zeros((b, n, cp - 3 - c), jnp.float32)
    return jnp.concatenate([xyz, feats, pad], axis=-1).reshape(b * n, cp)


def _sa(params, xyzT, tabf, ctrs, r2s, nsamples):
    idx, d32 = _select(xyzT, ctrs, k=32)
    b, k, m, _ = idx.shape
    rows = _sc_gather(tabf, idx.reshape(-1))
    g = rows.reshape(b, k, m, rows.shape[-1])
    return _mlpmax(g, d32, ctrs, params, r20=r2s[0], r21=r2s[1])


def kernel(points, params):
    pts = points.reshape(_B, _N, 5)
    xyz = pts[:, :, 1:4]
    feats = pts[:, :, 4:5]

    xyzT = _to_T(xyz)
    tab1 = _table(xyz, feats, 16)
    xyz1 = _fps(xyzT, 1024)
    f1 = _sa(params["sa1"], xyzT, tab1, xyz1, [0.25, 1.0], [16, 32])

    xyz1T = _to_T(xyz1)
    tab2 = _table(xyz1, f1, 80)
    xyz2 = _fps(xyz1T, 512)
    f2 = _sa(params["sa2"], xyz1T, tab2, xyz2, [1.0, 4.0], [16, 32])

    cls2 = _mlp(f2.reshape(-1, f2.shape[-1]), params["conf2"], last_linear=True)
    cls2 = cls2.reshape(_B, 512, 3)

    xyz2T = _to_T(xyz2)
    score = jnp.max(jax.nn.sigmoid(cls2), axis=-1)
    xyz3 = _topk_gather(score, xyz2T, 256)

    tab3 = _table(xyz2, f2, 144)
    f3 = _sa(params["sa3"], xyz2T, tab3, xyz3, [4.0, 16.0], [16, 32])

    cls3 = _mlp(f3.reshape(-1, f3.shape[-1]), params["conf3"], last_linear=True)
    cls3 = cls3.reshape(_B, 256, 3)

    offsets, centers = _vote(f3.reshape(-1, f3.shape[-1]),
                             xyz3.reshape(-1, 3),
                             params["vote_mlp"], params["vote_off"])
    offsets = offsets.reshape(_B, 256, 3)
    centers = centers.reshape(_B, 256, 3)

    xyz3T = _to_T(xyz3)
    tab4 = _table(xyz3, f3, 272)
    f4 = _sa(params["sa4"], xyz3T, tab4, centers, [16.0, 64.0], [16, 32])

    ctr_b = pts[:, :256, 0].reshape(-1, 1)
    centers_out = jnp.concatenate([ctr_b, centers.reshape(-1, 3)], axis=1)
    centers_origin_out = jnp.concatenate([ctr_b, xyz3.reshape(-1, 3)], axis=1)
    ctr_offsets_out = jnp.concatenate([ctr_b, offsets.reshape(-1, 3)], axis=1)
    centers_features = f4.reshape(-1, f4.shape[-1])
    return (centers_out, centers_origin_out, ctr_offsets_out,
            centers_features, cls2, cls3)
